# Initial kernel scaffold; baseline (speedup 1.0000x reference)
import jax
import jax.numpy as jnp
from jax.experimental import pallas as pl


def kernel(input, target):
    raise NotImplementedError("write your pallas kernel here")


# speedup vs baseline: 120.2531x; 120.2531x over previous
import functools

import jax
import jax.numpy as jnp
from jax import lax
from jax.experimental import pallas as pl
from jax.experimental.pallas import tpu as pltpu

_THRESH = 0.7
_MIN_KEPT = 10000

_ROWS = 8192
_COLS = 512
_BLOCK_ROWS = 1024


def _bce_terms(x, y):
    p = jax.nn.sigmoid(x)
    kept = ((y == 1.0) & (p <= _THRESH)) | ((y == 0.0) & (p >= 1.0 - _THRESH))
    per = jnp.maximum(x, 0.0) - x * y + jnp.log1p(jnp.exp(-jnp.abs(x)))
    return per, kept.astype(jnp.float32)


def _main_body(x_ref, y_ref, s_ref, n_ref):
    i = pl.program_id(0)
    per, kf = _bce_terms(x_ref[...], y_ref[...])

    @pl.when(i == 0)
    def _init():
        s_ref[...] = jnp.zeros((1, 1), jnp.float32)
        n_ref[...] = jnp.zeros((1, 1), jnp.float32)

    s_ref[...] += jnp.sum(per * kf).reshape(1, 1)
    n_ref[...] += jnp.sum(kf).reshape(1, 1)


@jax.jit
def _main_sums(x, y):
    grid = (_ROWS // _BLOCK_ROWS,)
    in_spec = pl.BlockSpec((_BLOCK_ROWS, _COLS), lambda i: (i, 0))
    out_spec = pl.BlockSpec((1, 1), lambda i: (0, 0))
    s, n = pl.pallas_call(
        _main_body,
        grid=grid,
        in_specs=[in_spec, in_spec],
        out_specs=[out_spec, out_spec],
        out_shape=[
            jax.ShapeDtypeStruct((1, 1), jnp.float32),
            jax.ShapeDtypeStruct((1, 1), jnp.float32),
        ],
    )(x, y)
    return s[0, 0], n[0, 0]


def _fallback_loss(ops):
    x, y, s, n = ops
    p = jax.nn.sigmoid(x)
    per, kf = _bce_terms(x, y)
    h = jnp.abs(p - 0.5).reshape(-1)
    _, idx = lax.top_k(-h, _MIN_KEPT)
    extra = 1.0 - kf.reshape(-1)[idx]
    s2 = s + jnp.sum(per.reshape(-1)[idx] * extra)
    n2 = n + jnp.sum(extra)
    return s2 / jnp.maximum(n2, 1.0)


def _main_loss(ops):
    _, _, s, n = ops
    return s / jnp.maximum(n, 1.0)


def kernel(input, target):
    x = input.reshape(_ROWS, _COLS)
    y = target.reshape(_ROWS, _COLS).astype(jnp.float32)
    s, n = _main_sums(x, y)
    return lax.cond(n < _MIN_KEPT, _fallback_loss, _main_loss, (x, y, s, n))
